# baseline (speedup 1.0000x reference)
import functools

import jax
import jax.numpy as jnp
from jax import lax
from jax.experimental import pallas as pl
from jax.experimental.pallas import tpu as pltpu
from jax.experimental.pallas import tpu_sc as plsc

N_NODES = 10000
D_FEAT = 128
N_EDGES = 320000

NC = 2
NS = 16
NW = NC * NS

CHUNK = 64
NCHUNK = 160
NBUF = 5
NCOLS = 2 * NBUF
Y_ROWS = N_NODES + 16
E_PER_W = CHUNK * NCHUNK
E_PAD = E_PER_W * NW

N_PAD = 10240
ACC_ROWS = 10112
ACC_PER_TILE = ACC_ROWS // NS

_MESH = plsc.VectorSubcoreMesh(
    core_axis_name="c", subcore_axis_name="s", num_cores=NC, num_subcores=NS)


@functools.partial(
    pl.kernel,
    out_type=jax.ShapeDtypeStruct((NW, N_PAD), jnp.float32),
    mesh=_MESH,
    compiler_params=pltpu.CompilerParams(needs_layout_passes=False),
    scratch_types=[
        pltpu.VMEM((E_PER_W,), jnp.int32),
        pltpu.VMEM((N_PAD,), jnp.float32),
    ],
)
def _sc_degree(row_hbm, out_hbm, rows_v, deg_v):
    c = lax.axis_index("c")
    s = lax.axis_index("s")
    wid = s * NC + c
    pltpu.sync_copy(row_hbm.at[wid], rows_v)

    def zbody(i, carry):
        deg_v[pl.ds(i * 16, 16)] = jnp.zeros((16,), jnp.float32)
        return carry

    lax.fori_loop(jnp.int32(0), jnp.int32(N_PAD // 16), zbody, jnp.int32(0))

    def body(k, carry):
        idx = rows_v[pl.ds(k * 16, 16)]
        plsc.addupdate_scatter(deg_v, [idx], jnp.ones((16,), jnp.float32))
        return carry

    lax.fori_loop(jnp.int32(0), jnp.int32(E_PER_W // 16), body, jnp.int32(0))
    pltpu.sync_copy(deg_v, out_hbm.at[wid])
    return None


@functools.partial(
    pl.kernel,
    out_type=jax.ShapeDtypeStruct((NC, ACC_ROWS, D_FEAT), jnp.float32),
    mesh=_MESH,
    scratch_types=(
        [pltpu.VMEM((CHUNK,), jnp.int32) for _ in range(NCOLS)]
        + [pltpu.VMEM((CHUNK,), jnp.int32) for _ in range(NBUF)]
        + [pltpu.VMEM((CHUNK, D_FEAT), jnp.float32) for _ in range(NBUF)]
        + [pltpu.VMEM_SHARED((ACC_ROWS, D_FEAT), jnp.float32)]
        + [pltpu.SemaphoreType.DMA for _ in range(NCOLS + 2 * NBUF)]
    ),
)
def _sc_aggregate(y_hbm, row_hbm, col_hbm, zeros_hbm, out_hbm, *scr):
    cbufs = scr[:NCOLS]
    rbufs = scr[NCOLS:NCOLS + NBUF]
    bufs = scr[NCOLS + NBUF:NCOLS + 2 * NBUF]
    acc_sh = scr[NCOLS + 2 * NBUF]
    csems = scr[NCOLS + 2 * NBUF + 1:2 * NCOLS + 2 * NBUF + 1]
    rsems = scr[2 * NCOLS + 2 * NBUF + 1:2 * NCOLS + 3 * NBUF + 1]
    sems = scr[2 * NCOLS + 3 * NBUF + 1:]

    c = lax.axis_index("c")
    s = lax.axis_index("s")
    wid = s * NC + c
    base = s * ACC_PER_TILE

    pltpu.sync_copy(zeros_hbm, acc_sh.at[pl.ds(base, ACC_PER_TILE)])
    plsc.subcore_barrier()

    for k in range(NCOLS):
        jk = jnp.int32(k)
        pltpu.async_copy(col_hbm.at[wid, pl.ds(jk * CHUNK, CHUNK)],
                         cbufs[k], csems[k])
    for b in range(NBUF):
        jb = jnp.int32(b)
        pltpu.async_copy(row_hbm.at[wid, pl.ds(jb * CHUNK, CHUNK)],
                         rbufs[b], rsems[b])
        pltpu.make_async_copy(col_hbm.at[wid, pl.ds(jb * CHUNK, CHUNK)],
                              cbufs[b], csems[b]).wait()
        pltpu.async_copy(y_hbm.at[cbufs[b]], bufs[b], sems[b])

    def body(g, carry):
        j0 = g * NCOLS
        for k in range(NCOLS):
            j = j0 + k
            b = k % NBUF
            pltpu.make_async_copy(y_hbm.at[cbufs[k]], bufs[b], sems[b]).wait()
            pltpu.make_async_copy(
                row_hbm.at[wid, pl.ds(j * CHUNK, CHUNK)],
                rbufs[b], rsems[b]).wait()
            pltpu.sync_copy(bufs[b], acc_sh.at[rbufs[b]], add=True)

            kn = (k + NBUF) % NCOLS

            @pl.when(j + NBUF < NCHUNK)
            def _():
                pltpu.async_copy(
                    row_hbm.at[wid, pl.ds((j + NBUF) * CHUNK, CHUNK)],
                    rbufs[b], rsems[b])
                pltpu.make_async_copy(
                    col_hbm.at[wid, pl.ds((j + NBUF) * CHUNK, CHUNK)],
                    cbufs[kn], csems[kn]).wait()
                pltpu.async_copy(y_hbm.at[cbufs[kn]], bufs[b], sems[b])

            @pl.when(j + NCOLS < NCHUNK)
            def _():
                pltpu.async_copy(
                    col_hbm.at[wid, pl.ds((j + NCOLS) * CHUNK, CHUNK)],
                    cbufs[k], csems[k])

        return carry

    lax.fori_loop(jnp.int32(0), jnp.int32(NCHUNK // NCOLS), body, jnp.int32(0))

    plsc.subcore_barrier()
    pltpu.sync_copy(acc_sh.at[pl.ds(base, ACC_PER_TILE)],
                    out_hbm.at[c, pl.ds(base, ACC_PER_TILE)])
    return None


def _dis_from_parts(deg_parts):
    deg = jnp.sum(deg_parts, axis=0)[:N_NODES, None]
    return jnp.where(deg > 0, lax.rsqrt(jnp.maximum(deg, 1e-38)), 0.0)


def _tc_prep_body(deg_ref, xp_ref, y_ref):
    deg = jnp.sum(deg_ref[...], axis=0)[:Y_ROWS, None]
    rid = lax.broadcasted_iota(jnp.int32, (Y_ROWS, 1), 0)
    dis = jnp.where((deg > 0) & (rid < N_NODES),
                    lax.rsqrt(jnp.maximum(deg, 1e-38)), 0.0)
    y_ref[...] = dis * xp_ref[...]


def _tc_final_body(deg_ref, x_ref, acc_ref, out_ref):
    dis = _dis_from_parts(deg_ref[...])
    acc = acc_ref[...]
    out_ref[:, :D_FEAT] = x_ref[...]
    out_ref[:, D_FEAT:] = dis * (acc[0, :N_NODES] + acc[1, :N_NODES])


_tc_prep = pl.pallas_call(
    _tc_prep_body,
    out_shape=jax.ShapeDtypeStruct((Y_ROWS, D_FEAT), jnp.float32),
)

_tc_final = pl.pallas_call(
    _tc_final_body,
    out_shape=jax.ShapeDtypeStruct((N_NODES, 2 * D_FEAT), jnp.float32),
)


@jax.jit
def _run(x, edge_index):
    row = edge_index[0].astype(jnp.int32)
    col = edge_index[1].astype(jnp.int32)
    pad = E_PAD - N_EDGES
    drow_deg = N_NODES + (jnp.arange(pad, dtype=jnp.int32) % (N_PAD - N_NODES))
    drow_agg = jnp.arange(pad, dtype=jnp.int32) % N_NODES
    dcol = N_NODES + (jnp.arange(pad, dtype=jnp.int32) % (Y_ROWS - N_NODES))
    rowd_p = jnp.concatenate([row, drow_deg]).reshape(NW, E_PER_W)
    row_p = jnp.concatenate([row, drow_agg]).reshape(NW, E_PER_W)
    col_p = jnp.concatenate([col, dcol]).reshape(NW, E_PER_W)

    z128 = jnp.zeros((ACC_PER_TILE, D_FEAT), jnp.float32)

    xp = jnp.pad(x, ((0, Y_ROWS - N_NODES), (0, 0)))
    deg_parts = _sc_degree(rowd_p)
    y = _tc_prep(deg_parts, xp)
    acc_parts = _sc_aggregate(y, row_p, col_p, z128)
    return _tc_final(deg_parts, x, acc_parts)


def kernel(x, edge_index):
    return _run(x, edge_index)


# speedup vs baseline: 1.0380x; 1.0380x over previous
import functools

import jax
import jax.numpy as jnp
from jax import lax
from jax.experimental import pallas as pl
from jax.experimental.pallas import tpu as pltpu
from jax.experimental.pallas import tpu_sc as plsc

N_NODES = 10000
D_FEAT = 128
N_EDGES = 320000

NC = 2
NS = 16
NW = NC * NS

CHUNK = 64
NCHUNK = 160
NBUF = 4

E_PER_W = CHUNK * NCHUNK
E_PAD = E_PER_W * NW

N_PAD = 10240
ROWS_PER_TILE = N_PAD // NS

_MESH = plsc.VectorSubcoreMesh(
    core_axis_name="c", subcore_axis_name="s", num_cores=NC, num_subcores=NS)


@functools.partial(
    pl.kernel,
    out_type=jax.ShapeDtypeStruct((NW, N_PAD), jnp.float32),
    mesh=_MESH,
    compiler_params=pltpu.CompilerParams(needs_layout_passes=False),
    scratch_types=[
        pltpu.VMEM((E_PER_W,), jnp.int32),
        pltpu.VMEM((N_PAD,), jnp.float32),
    ],
)
def _sc_degree(row_hbm, out_hbm, rows_v, deg_v):
    c = lax.axis_index("c")
    s = lax.axis_index("s")
    wid = s * NC + c
    pltpu.sync_copy(row_hbm.at[wid], rows_v)

    def zbody(i, carry):
        deg_v[pl.ds(i * 16, 16)] = jnp.zeros((16,), jnp.float32)
        return carry

    lax.fori_loop(jnp.int32(0), jnp.int32(N_PAD // 16), zbody, jnp.int32(0))

    def body(k, carry):
        idx = rows_v[pl.ds(k * 16, 16)]
        plsc.addupdate_scatter(deg_v, [idx], jnp.ones((16,), jnp.float32))
        return carry

    lax.fori_loop(jnp.int32(0), jnp.int32(E_PER_W // 16), body, jnp.int32(0))
    pltpu.sync_copy(deg_v, out_hbm.at[wid])
    return None


@functools.partial(
    pl.kernel,
    out_type=jax.ShapeDtypeStruct((NC, N_PAD, D_FEAT), jnp.float32),
    mesh=_MESH,
    scratch_types=(
        [pltpu.VMEM((E_PER_W,), jnp.int32)]
        + [pltpu.VMEM((CHUNK,), jnp.int32) for _ in range(NBUF)]
        + [pltpu.VMEM((CHUNK, D_FEAT), jnp.float32) for _ in range(NBUF)]
        + [pltpu.VMEM_SHARED((N_PAD, D_FEAT), jnp.float32)]
        + [pltpu.SemaphoreType.DMA for _ in range(2 * NBUF)]
    ),
)
def _sc_aggregate(y_hbm, row_hbm, col_hbm, zeros_hbm, out_hbm,
                  cols_v, *scr):
    rbufs = scr[:NBUF]
    bufs = scr[NBUF:2 * NBUF]
    acc_sh = scr[2 * NBUF]
    rsems = scr[2 * NBUF + 1:2 * NBUF + 1 + NBUF]
    sems = scr[2 * NBUF + 1 + NBUF:]

    c = lax.axis_index("c")
    s = lax.axis_index("s")
    wid = s * NC + c
    base = s * ROWS_PER_TILE

    pltpu.sync_copy(zeros_hbm, acc_sh.at[pl.ds(base, ROWS_PER_TILE)])
    pltpu.sync_copy(col_hbm.at[wid], cols_v)
    plsc.subcore_barrier()

    for b in range(NBUF):
        jb = jnp.int32(b)
        pltpu.async_copy(row_hbm.at[wid, pl.ds(jb * CHUNK, CHUNK)],
                         rbufs[b], rsems[b])
        pltpu.async_copy(y_hbm.at[cols_v.at[pl.ds(jb * CHUNK, CHUNK)]],
                         bufs[b], sems[b])

    def body(g, carry):
        j0 = g * NBUF
        for b in range(NBUF):
            j = j0 + b
            pltpu.make_async_copy(
                y_hbm.at[cols_v.at[pl.ds(j * CHUNK, CHUNK)]],
                bufs[b], sems[b]).wait()
            pltpu.make_async_copy(
                row_hbm.at[wid, pl.ds(j * CHUNK, CHUNK)],
                rbufs[b], rsems[b]).wait()
            pltpu.sync_copy(bufs[b], acc_sh.at[rbufs[b]], add=True)

            @pl.when(j + NBUF < NCHUNK)
            def _():
                pltpu.async_copy(
                    row_hbm.at[wid, pl.ds((j + NBUF) * CHUNK, CHUNK)],
                    rbufs[b], rsems[b])
                pltpu.async_copy(
                    y_hbm.at[cols_v.at[pl.ds((j + NBUF) * CHUNK, CHUNK)]],
                    bufs[b], sems[b])

        return carry

    lax.fori_loop(jnp.int32(0), jnp.int32(NCHUNK // NBUF), body, jnp.int32(0))

    plsc.subcore_barrier()
    pltpu.sync_copy(acc_sh.at[pl.ds(base, ROWS_PER_TILE)],
                    out_hbm.at[c, pl.ds(base, ROWS_PER_TILE)])
    return None


def _dis_from_parts(deg_parts):
    deg = jnp.sum(deg_parts, axis=0)[:N_NODES, None]
    return jnp.where(deg > 0, lax.rsqrt(jnp.maximum(deg, 1e-38)), 0.0)


def _tc_prep_body(deg_ref, x_ref, y_ref):
    y_ref[...] = _dis_from_parts(deg_ref[...]) * x_ref[...]


def _tc_final_body(deg_ref, x_ref, acc_ref, out_ref):
    dis = _dis_from_parts(deg_ref[...])
    acc = acc_ref[...]
    out_ref[:, :D_FEAT] = x_ref[...]
    out_ref[:, D_FEAT:] = dis * (acc[0, :N_NODES] + acc[1, :N_NODES])


_tc_prep = pl.pallas_call(
    _tc_prep_body,
    out_shape=jax.ShapeDtypeStruct((N_NODES, D_FEAT), jnp.float32),
)

_tc_final = pl.pallas_call(
    _tc_final_body,
    out_shape=jax.ShapeDtypeStruct((N_NODES, 2 * D_FEAT), jnp.float32),
)


@jax.jit
def _run(x, edge_index):
    row = edge_index[0].astype(jnp.int32)
    col = edge_index[1].astype(jnp.int32)
    pad = E_PAD - N_EDGES
    drow = N_NODES + (jnp.arange(pad, dtype=jnp.int32) % (N_PAD - N_NODES))
    dcol = jnp.arange(pad, dtype=jnp.int32) % N_NODES
    row_p = jnp.concatenate([row, drow]).reshape(NW, E_PER_W)
    col_p = jnp.concatenate([col, dcol]).reshape(NW, E_PER_W)

    z128 = jnp.zeros((ROWS_PER_TILE, D_FEAT), jnp.float32)

    deg_parts = _sc_degree(row_p)
    y = _tc_prep(deg_parts, x)
    acc_parts = _sc_aggregate(y, row_p, col_p, z128)
    return _tc_final(deg_parts, x, acc_parts)


def kernel(x, edge_index):
    return _run(x, edge_index)
